# baseline (speedup 1.0000x reference)
import functools

import jax
import jax.numpy as jnp
from jax import lax
from jax.experimental import pallas as pl
from jax.experimental.pallas import tpu as pltpu
import jax.experimental.pallas.tpu_sc as plsc

E_TOTAL = 160000
FDIM = 256
NPROBE_OUT = 10000

NCORE = 2
NSUB = 16
NWORK = NCORE * NSUB
CHUNK = 128
ACC = 10240
SLICE = ACC // NSUB

BLK = 2048
BROWS = BLK // CHUNK
NMAIN = E_TOTAL // BLK
NA = 39
E_HALF = NA * BLK
E_MAIN = NMAIN * BLK
TAIL = E_TOTAL - E_MAIN
TPAD = 4096
TROWS = TPAD // CHUNK


def _mlp_compute(x, wall, b1, i1, b2, i2, b3, i3, wo, bo):
    for k, (b_ref, idt_ref) in enumerate(((b1, i1), (b2, i2), (b3, i3))):
        hh = jnp.dot(x, wall[k * FDIM:(k + 1) * FDIM],
                     preferred_element_type=jnp.float32)
        hh = hh + b_ref[...] * 0.5
        q = hh * idt_ref[...]
        x = x + q + q * jnp.tanh(hh)
    s_wide = jnp.dot(x, wo[...], preferred_element_type=jnp.float32)
    n = x.shape[0] // CHUNK
    s3 = s_wide.reshape(n, CHUNK, CHUNK)
    sub = lax.broadcasted_iota(jnp.int32, (n, CHUNK, CHUNK), 1)
    lane = lax.broadcasted_iota(jnp.int32, (n, CHUNK, CHUNK), 2)
    return jnp.sum(jnp.where(sub == lane, s3, 0.0), axis=1) + bo[0, 0]


def _mlp_body(x_ref, wall, b1, i1, b2, i2, b3, i3, wo, bo, out_ref):
    out_ref[...] = _mlp_compute(x_ref[...], wall, b1, i1, b2, i2, b3, i3,
                                wo, bo)


def _tail_body(x_ref, wall, b1, i1, b2, i2, b3, i3, wo, bo, out_ref):
    s2 = _mlp_compute(x_ref[...], wall, b1, i1, b2, i2, b3, i3, wo, bo)
    out_ref[...] = jnp.concatenate(
        [s2, jnp.zeros((TROWS - TAIL // CHUNK, CHUNK), jnp.float32)], axis=0)


def _edge_mlp(ef, *wargs):
    full2 = lambda shape: pl.BlockSpec(shape, lambda i: (0, 0))
    row = full2((1, FDIM))
    wspecs = [
        full2((3 * FDIM, FDIM)), row, row, row, row, row, row,
        full2((FDIM, CHUNK)), full2((1, 1)),
    ]

    def half(off):
        return pl.pallas_call(
            _mlp_body,
            grid=(NA,),
            in_specs=[
                pl.BlockSpec((BLK, FDIM), lambda i: (i + off, 0)),
            ] + wspecs,
            out_specs=pl.BlockSpec((BROWS, CHUNK), lambda i: (i, 0)),
            out_shape=jax.ShapeDtypeStruct((E_HALF // CHUNK, CHUNK),
                                           jnp.float32),
        )(ef, *wargs)

    main_a = half(0)
    main_b = half(NA)
    tail = pl.pallas_call(
        _tail_body,
        grid=(1,),
        in_specs=[
            pl.BlockSpec((TAIL, FDIM), lambda i: (E_MAIN // TAIL, 0)),
        ] + wspecs,
        out_specs=pl.BlockSpec((TROWS, CHUNK), lambda i: (0, 0)),
        out_shape=jax.ShapeDtypeStruct((TROWS, CHUNK), jnp.float32),
    )(ef, *wargs)
    return main_a, main_b, tail


@functools.cache
def _make_scatter(segs):
    per = sum(c for _, c in segs)
    mesh = plsc.VectorSubcoreMesh(
        core_axis_name="c", subcore_axis_name="s", num_cores=NCORE)

    @functools.partial(
        pl.kernel,
        out_type=jax.ShapeDtypeStruct((NCORE * ACC,), jnp.float32),
        mesh=mesh,
        scratch_types=[
            pltpu.VMEM((per,), jnp.int32),
            pltpu.VMEM((per,), jnp.float32),
            pltpu.VMEM((per,), jnp.float32),
            pltpu.VMEM((SLICE,), jnp.float32),
            pltpu.VMEM_SHARED((ACC,), jnp.float32),
            pltpu.SemaphoreType.DMA,
        ],
    )
    def scatter(*args):
        ops = args[:3 * len(segs)]
        (out_hbm, idx_v, val_v, ew_v, zbuf, acc_sh,
         stage_sem) = args[3 * len(segs):]
        cid = lax.axis_index("c")
        sid = lax.axis_index("s")
        wid = sid * NCORE + cid
        copies = []
        off = 0
        for k, (goff, cnt) in enumerate(segs):
            dst = pl.ds(off, cnt)
            copies.append(pltpu.async_copy(
                ops[3 * k].at[pl.ds(goff + wid * cnt, cnt)], idx_v.at[dst],
                stage_sem))
            copies.append(pltpu.async_copy(
                ops[3 * k + 1].at[pl.ds(wid * cnt, cnt)], val_v.at[dst],
                stage_sem))
            copies.append(pltpu.async_copy(
                ops[3 * k + 2].at[pl.ds(goff + wid * cnt, cnt)], ew_v.at[dst],
                stage_sem))
            off += cnt
        def zbody(i, c):
            zbuf[pl.ds(i * 16, 16)] = jnp.zeros((16,), jnp.float32)
            return c
        lax.fori_loop(0, SLICE // 16, zbody, 0)
        pltpu.sync_copy(zbuf, acc_sh.at[pl.ds(sid * SLICE, SLICE)])
        for c in copies:
            c.wait()
        def wbody(i, c):
            d = pl.ds(i * 16, 16)
            val_v[d] = val_v[d] * ew_v[d]
            return c
        lax.fori_loop(0, per // 16, wbody, 0)
        plsc.subcore_barrier()
        pltpu.sync_copy(val_v, acc_sh.at[idx_v], add=True)
        plsc.subcore_barrier()
        pltpu.sync_copy(acc_sh.at[pl.ds(sid * SLICE, SLICE)],
                        out_hbm.at[pl.ds(cid * ACC + sid * SLICE, SLICE)])

    return scatter


def kernel(edge_features, node_probe, edge_weight, nprobe,
           W1, b1, idt1, W2, b2, idt2, W3, b3, idt3, Wout, bout):
    row = lambda v: v.reshape(1, FDIM)
    wall = 0.5 * jnp.concatenate([W1, W2, W3], axis=0)
    main_a, main_b, tail = _edge_mlp(
        edge_features, wall,
        row(b1), row(idt1), row(b2), row(idt2), row(b3), row(idt3),
        jnp.tile(Wout, (1, CHUNK)), bout.reshape(1, 1))
    ew = edge_weight[:, 0]
    idx_t = jnp.pad(node_probe[E_MAIN:], (0, TPAD - TAIL))
    ew_t = jnp.pad(ew[E_MAIN:], (0, TPAD - TAIL))
    acc_a = _make_scatter(((0, E_HALF // NWORK),))(
        node_probe, main_a.reshape(E_HALF), ew)
    acc_b = _make_scatter(((E_HALF, E_HALF // NWORK),
                           (0, TPAD // NWORK)))(
        node_probe, main_b.reshape(E_HALF), ew,
        idx_t, tail.reshape(TPAD), ew_t)
    acc = (acc_a[:ACC] + acc_a[ACC:]) + (acc_b[:ACC] + acc_b[ACC:])
    return acc[:NPROBE_OUT]


# speedup vs baseline: 1.0066x; 1.0066x over previous
import functools

import jax
import jax.numpy as jnp
from jax import lax
from jax.experimental import pallas as pl
from jax.experimental.pallas import tpu as pltpu
import jax.experimental.pallas.tpu_sc as plsc

E_TOTAL = 160000
FDIM = 256
NPROBE_OUT = 10000

NCORE = 2
NSUB = 16
NWORK = NCORE * NSUB
CHUNK = 128
ACC = 10240
SLICE = ACC // NSUB

BLK = 2048
BROWS = BLK // CHUNK
NMAIN = E_TOTAL // BLK
NA = 39
E_HALF = NA * BLK
E_MAIN = NMAIN * BLK
TAIL = E_TOTAL - E_MAIN
TPAD = 4096
TROWS = TPAD // CHUNK


def _mlp_compute(x, wall, b1, i1, b2, i2, b3, i3, wo, bo):
    for k, (b_ref, idt_ref) in enumerate(((b1, i1), (b2, i2), (b3, i3))):
        hh = jnp.dot(x, wall[k * FDIM:(k + 1) * FDIM],
                     preferred_element_type=jnp.float32)
        hh = hh + b_ref[...] * 0.5
        q = hh * idt_ref[...]
        x = x + q + q * jnp.tanh(hh)
    s_wide = jnp.dot(x, wo[...], preferred_element_type=jnp.float32)
    n = x.shape[0] // CHUNK
    s3 = s_wide.reshape(n, CHUNK, CHUNK)
    sub = lax.broadcasted_iota(jnp.int32, (n, CHUNK, CHUNK), 1)
    lane = lax.broadcasted_iota(jnp.int32, (n, CHUNK, CHUNK), 2)
    return jnp.sum(jnp.where(sub == lane, s3, 0.0), axis=1) + bo[0, 0]


def _mlp_body(x_ref, ew_ref, wall, b1, i1, b2, i2, b3, i3, wo, bo, out_ref):
    s2 = _mlp_compute(x_ref[...], wall, b1, i1, b2, i2, b3, i3, wo, bo)
    out_ref[...] = s2 * ew_ref[...]


def _tail_body(x_ref, ew_ref, wall, b1, i1, b2, i2, b3, i3, wo, bo, out_ref):
    s2 = _mlp_compute(x_ref[...], wall, b1, i1, b2, i2, b3, i3, wo, bo)
    s2 = s2 * ew_ref[0:TAIL // CHUNK]
    out_ref[...] = jnp.concatenate(
        [s2, jnp.zeros((TROWS - TAIL // CHUNK, CHUNK), jnp.float32)], axis=0)


def _edge_mlp(ef, ew2d, *wargs):
    full2 = lambda shape: pl.BlockSpec(shape, lambda i: (0, 0))
    row = full2((1, FDIM))
    wspecs = [
        full2((3 * FDIM, FDIM)), row, row, row, row, row, row,
        full2((FDIM, CHUNK)), full2((1, 1)),
    ]

    def half(off):
        return pl.pallas_call(
            _mlp_body,
            grid=(NA,),
            in_specs=[
                pl.BlockSpec((BLK, FDIM), lambda i: (i + off, 0)),
                pl.BlockSpec((BROWS, CHUNK), lambda i: (i + off, 0)),
            ] + wspecs,
            out_specs=pl.BlockSpec((BROWS, CHUNK), lambda i: (i, 0)),
            out_shape=jax.ShapeDtypeStruct((E_HALF // CHUNK, CHUNK),
                                           jnp.float32),
        )(ef, ew2d, *wargs)

    main_a = half(0)
    main_b = half(NA)
    tail = pl.pallas_call(
        _tail_body,
        grid=(1,),
        in_specs=[
            pl.BlockSpec((TAIL, FDIM), lambda i: (E_MAIN // TAIL, 0)),
            pl.BlockSpec((TROWS, CHUNK), lambda i: (E_MAIN // CHUNK // TROWS,
                                                    0)),
        ] + wspecs,
        out_specs=pl.BlockSpec((TROWS, CHUNK), lambda i: (0, 0)),
        out_shape=jax.ShapeDtypeStruct((TROWS, CHUNK), jnp.float32),
    )(ef, ew2d, *wargs)
    return main_a, main_b, tail


@functools.cache
def _make_scatter(segs):
    per = sum(c for _, c in segs)
    mesh = plsc.VectorSubcoreMesh(
        core_axis_name="c", subcore_axis_name="s", num_cores=NCORE)

    @functools.partial(
        pl.kernel,
        out_type=jax.ShapeDtypeStruct((NCORE * ACC,), jnp.float32),
        mesh=mesh,
        scratch_types=[
            pltpu.VMEM((per,), jnp.int32),
            pltpu.VMEM((per,), jnp.float32),
            pltpu.VMEM((SLICE,), jnp.float32),
            pltpu.VMEM_SHARED((ACC,), jnp.float32),
            pltpu.SemaphoreType.DMA,
        ],
    )
    def scatter(*args):
        ops = args[:2 * len(segs)]
        out_hbm, idx_v, val_v, zbuf, acc_sh, stage_sem = args[2 * len(segs):]
        cid = lax.axis_index("c")
        sid = lax.axis_index("s")
        wid = sid * NCORE + cid
        copies = []
        off = 0
        for k, (goff, cnt) in enumerate(segs):
            dst = pl.ds(off, cnt)
            copies.append(pltpu.async_copy(
                ops[2 * k].at[pl.ds(goff + wid * cnt, cnt)], idx_v.at[dst],
                stage_sem))
            copies.append(pltpu.async_copy(
                ops[2 * k + 1].at[pl.ds(wid * cnt, cnt)], val_v.at[dst],
                stage_sem))
            off += cnt
        def zbody(i, c):
            zbuf[pl.ds(i * 16, 16)] = jnp.zeros((16,), jnp.float32)
            return c
        lax.fori_loop(0, SLICE // 16, zbody, 0)
        pltpu.sync_copy(zbuf, acc_sh.at[pl.ds(sid * SLICE, SLICE)])
        for c in copies:
            c.wait()
        plsc.subcore_barrier()
        pltpu.sync_copy(val_v, acc_sh.at[idx_v], add=True)
        plsc.subcore_barrier()
        pltpu.sync_copy(acc_sh.at[pl.ds(sid * SLICE, SLICE)],
                        out_hbm.at[pl.ds(cid * ACC + sid * SLICE, SLICE)])

    return scatter


def kernel(edge_features, node_probe, edge_weight, nprobe,
           W1, b1, idt1, W2, b2, idt2, W3, b3, idt3, Wout, bout):
    row = lambda v: v.reshape(1, FDIM)
    ew2d = jnp.pad(edge_weight[:, 0],
                   (0, E_MAIN + TPAD - E_TOTAL)).reshape(-1, CHUNK)
    wall = 0.5 * jnp.concatenate([W1, W2, W3], axis=0)
    main_a, main_b, tail = _edge_mlp(
        edge_features, ew2d, wall,
        row(b1), row(idt1), row(b2), row(idt2), row(b3), row(idt3),
        jnp.tile(Wout, (1, CHUNK)), bout.reshape(1, 1))
    idx_t = jnp.pad(node_probe[E_MAIN:], (0, TPAD - TAIL))
    acc_a = _make_scatter(((0, E_HALF // NWORK),))(
        node_probe, main_a.reshape(E_HALF))
    acc_b = _make_scatter(((E_HALF, E_HALF // NWORK),
                           (0, TPAD // NWORK)))(
        node_probe, main_b.reshape(E_HALF), idx_t, tail.reshape(TPAD))
    acc = (acc_a[:ACC] + acc_a[ACC:]) + (acc_b[:ACC] + acc_b[ACC:])
    return acc[:NPROBE_OUT]
